# baseline (speedup 1.0000x reference)
import functools

import jax
import jax.numpy as jnp
from jax import lax
from jax.experimental import pallas as pl
from jax.experimental.pallas import tpu as pltpu
from jax.experimental.pallas import tpu_sc as plsc

H = W = 384
C = 384
B = H * W
NC, NS = 2, 16
NW = NC * NS
PPW = B // NW
K = 32
NCH = PPW // K


def _prep_body(u_ref, v_ref, i00, i10, i01, i11, w00, w10, w01, w11):
    u = u_ref[...]
    v = v_ref[...]
    gx = u * 2.0 - 1.0
    gy = -(v * 2.0 - 1.0)
    ix = ((gx + 1.0) * W - 1.0) * 0.5
    iy = ((gy + 1.0) * H - 1.0) * 0.5
    ix0 = jnp.floor(ix)
    iy0 = jnp.floor(iy)
    ix1 = ix0 + 1.0
    iy1 = iy0 + 1.0
    wx1 = ix - ix0
    wx0 = 1.0 - wx1
    wy1 = iy - iy0
    wy0 = 1.0 - wy1
    vx0 = (ix0 >= 0) & (ix0 <= W - 1)
    vx1 = (ix1 >= 0) & (ix1 <= W - 1)
    vy0 = (iy0 >= 0) & (iy0 <= H - 1)
    vy1 = (iy1 >= 0) & (iy1 <= H - 1)
    cx0 = jnp.clip(ix0, 0, W - 1).astype(jnp.int32)
    cx1 = jnp.clip(ix1, 0, W - 1).astype(jnp.int32)
    cy0 = jnp.clip(iy0, 0, H - 1).astype(jnp.int32)
    cy1 = jnp.clip(iy1, 0, H - 1).astype(jnp.int32)
    live = u != 0.0
    zero = jnp.zeros_like(u)
    i00[...] = cy0 * W + cx0
    i10[...] = cy0 * W + cx1
    i01[...] = cy1 * W + cx0
    i11[...] = cy1 * W + cx1
    w00[...] = jnp.where(vx0 & vy0 & live, wx0 * wy0, zero)
    w10[...] = jnp.where(vx1 & vy0 & live, wx1 * wy0, zero)
    w01[...] = jnp.where(vx0 & vy1 & live, wx0 * wy1, zero)
    w11[...] = jnp.where(vx1 & vy1 & live, wx1 * wy1, zero)


def _prep(u, v):
    shp = jax.ShapeDtypeStruct((H, W), jnp.int32)
    shpf = jax.ShapeDtypeStruct((H, W), jnp.float32)
    return pl.pallas_call(
        _prep_body,
        out_shape=(shp, shp, shp, shp, shpf, shpf, shpf, shpf),
    )(u, v)


def _tr_body(in_ref, out_ref):
    out_ref[...] = in_ref[...].T


def _transpose_in(tex2):
    bw = 2048
    return pl.pallas_call(
        _tr_body,
        grid=(B // bw,),
        in_specs=[pl.BlockSpec((C, bw), lambda j: (0, j))],
        out_specs=pl.BlockSpec((bw, C), lambda j: (j, 0)),
        out_shape=jax.ShapeDtypeStruct((B, C), jnp.float32),
    )(tex2)


def _transpose_out(rows):
    bw = 2048
    return pl.pallas_call(
        _tr_body,
        grid=(B // bw,),
        in_specs=[pl.BlockSpec((bw, C), lambda j: (j, 0))],
        out_specs=pl.BlockSpec((C, bw), lambda j: (0, j)),
        out_shape=jax.ShapeDtypeStruct((C, B), jnp.float32),
    )(rows)


def _sc_body(table, i00h, i10h, i01h, i11h, w00h, w10h, w01h, w11h, out_hbm,
             i00v, i10v, i01v, i11v, w00v, w10v, w01v, w11v,
             r00, r10, r01, r11, outv, sem):
    wid = lax.axis_index("s") * NC + lax.axis_index("c")
    base = wid * PPW

    def chunk(g, carry):
        pix = pl.multiple_of(base + g * K, K)
        pltpu.sync_copy(i00h.at[pl.ds(pix, K)], i00v)
        pltpu.sync_copy(i10h.at[pl.ds(pix, K)], i10v)
        pltpu.sync_copy(i01h.at[pl.ds(pix, K)], i01v)
        pltpu.sync_copy(i11h.at[pl.ds(pix, K)], i11v)
        pltpu.sync_copy(w00h.at[pl.ds(pix, K)], w00v)
        pltpu.sync_copy(w10h.at[pl.ds(pix, K)], w10v)
        pltpu.sync_copy(w01h.at[pl.ds(pix, K)], w01v)
        pltpu.sync_copy(w11h.at[pl.ds(pix, K)], w11v)
        c0 = pltpu.async_copy(table.at[i00v], r00, sem)
        c1 = pltpu.async_copy(table.at[i10v], r10, sem)
        c2 = pltpu.async_copy(table.at[i01v], r01, sem)
        c3 = pltpu.async_copy(table.at[i11v], r11, sem)
        c0.wait()
        c1.wait()
        c2.wait()
        c3.wait()

        def grp_body(g2, carry2):
            off = g2 * 16
            wv00 = w00v[pl.ds(off, 16)]
            wv10 = w10v[pl.ds(off, 16)]
            wv01 = w01v[pl.ds(off, 16)]
            wv11 = w11v[pl.ds(off, 16)]
            for l in range(16):
                k = off + l
                w00b = jnp.full((16,), wv00[l], jnp.float32)
                w10b = jnp.full((16,), wv10[l], jnp.float32)
                w01b = jnp.full((16,), wv01[l], jnp.float32)
                w11b = jnp.full((16,), wv11[l], jnp.float32)
                for j in range(C // 16):
                    sl = pl.ds(16 * j, 16)
                    outv[k, sl] = (r00[k, sl] * w00b + r10[k, sl] * w10b
                                   + r01[k, sl] * w01b + r11[k, sl] * w11b)
            return carry2

        lax.fori_loop(0, K // 16, grp_body, 0)
        pltpu.sync_copy(outv, out_hbm.at[pl.ds(pix, K)])
        return carry

    lax.fori_loop(0, NCH, chunk, 0)


@functools.lru_cache(maxsize=1)
def _sc_gather():
  return functools.partial(
    pl.kernel,
    out_type=jax.ShapeDtypeStruct((B, C), jnp.float32),
    mesh=plsc.VectorSubcoreMesh(core_axis_name="c", subcore_axis_name="s",
                                num_cores=NC, num_subcores=NS),
    scratch_types=[
        pltpu.VMEM((K,), jnp.int32),
        pltpu.VMEM((K,), jnp.int32),
        pltpu.VMEM((K,), jnp.int32),
        pltpu.VMEM((K,), jnp.int32),
        pltpu.VMEM((K,), jnp.float32),
        pltpu.VMEM((K,), jnp.float32),
        pltpu.VMEM((K,), jnp.float32),
        pltpu.VMEM((K,), jnp.float32),
        pltpu.VMEM((K, C), jnp.float32),
        pltpu.VMEM((K, C), jnp.float32),
        pltpu.VMEM((K, C), jnp.float32),
        pltpu.VMEM((K, C), jnp.float32),
        pltpu.VMEM((K, C), jnp.float32),
        pltpu.SemaphoreType.DMA,
    ],
  )(_sc_body)


def kernel(uv_map, neural_tex):
    u = uv_map[0, :, :, 0]
    v = uv_map[0, :, :, 1]
    i00, i10, i01, i11, w00, w10, w01, w11 = _prep(u, v)
    table = _transpose_in(neural_tex.reshape(C, B))
    rows = _sc_gather()(table,
                      i00.reshape(B), i10.reshape(B),
                      i01.reshape(B), i11.reshape(B),
                      w00.reshape(B), w10.reshape(B),
                      w01.reshape(B), w11.reshape(B))
    out = _transpose_out(rows)
    return out.reshape(1, C, H, W)


# speedup vs baseline: 2.1419x; 2.1419x over previous
import functools

import jax
import jax.numpy as jnp
from jax import lax
from jax.experimental import pallas as pl
from jax.experimental.pallas import tpu as pltpu
from jax.experimental.pallas import tpu_sc as plsc

H = W = 384
C = 384
B = H * W
NC, NS = 2, 16
NW = NC * NS
PPW = B // NW
K = 16
NCH = PPW // K


def _prep_body(u_ref, v_ref, i00, i10, i01, i11, w00, w10, w01, w11):
    u = u_ref[...]
    v = v_ref[...]
    gx = u * 2.0 - 1.0
    gy = -(v * 2.0 - 1.0)
    ix = ((gx + 1.0) * W - 1.0) * 0.5
    iy = ((gy + 1.0) * H - 1.0) * 0.5
    ix0 = jnp.floor(ix)
    iy0 = jnp.floor(iy)
    ix1 = ix0 + 1.0
    iy1 = iy0 + 1.0
    wx1 = ix - ix0
    wx0 = 1.0 - wx1
    wy1 = iy - iy0
    wy0 = 1.0 - wy1
    vx0 = (ix0 >= 0) & (ix0 <= W - 1)
    vx1 = (ix1 >= 0) & (ix1 <= W - 1)
    vy0 = (iy0 >= 0) & (iy0 <= H - 1)
    vy1 = (iy1 >= 0) & (iy1 <= H - 1)
    cx0 = jnp.clip(ix0, 0, W - 1).astype(jnp.int32)
    cx1 = jnp.clip(ix1, 0, W - 1).astype(jnp.int32)
    cy0 = jnp.clip(iy0, 0, H - 1).astype(jnp.int32)
    cy1 = jnp.clip(iy1, 0, H - 1).astype(jnp.int32)
    live = u != 0.0
    zero = jnp.zeros_like(u)
    i00[...] = cy0 * W + cx0
    i10[...] = cy0 * W + cx1
    i01[...] = cy1 * W + cx0
    i11[...] = cy1 * W + cx1
    w00[...] = jnp.where(vx0 & vy0 & live, wx0 * wy0, zero)
    w10[...] = jnp.where(vx1 & vy0 & live, wx1 * wy0, zero)
    w01[...] = jnp.where(vx0 & vy1 & live, wx0 * wy1, zero)
    w11[...] = jnp.where(vx1 & vy1 & live, wx1 * wy1, zero)


def _prep(u, v):
    shp = jax.ShapeDtypeStruct((H, W), jnp.int32)
    shpf = jax.ShapeDtypeStruct((H, W), jnp.float32)
    return pl.pallas_call(
        _prep_body,
        out_shape=(shp, shp, shp, shp, shpf, shpf, shpf, shpf),
    )(u, v)


def _tr_body(in_ref, out_ref):
    out_ref[...] = in_ref[...].T


def _transpose_in(tex2):
    bw = 2048
    return pl.pallas_call(
        _tr_body,
        grid=(B // bw,),
        in_specs=[pl.BlockSpec((C, bw), lambda j: (0, j))],
        out_specs=pl.BlockSpec((bw, C), lambda j: (j, 0)),
        out_shape=jax.ShapeDtypeStruct((B, C), jnp.float32),
    )(tex2)


def _transpose_out(rows):
    bw = 2048
    return pl.pallas_call(
        _tr_body,
        grid=(B // bw,),
        in_specs=[pl.BlockSpec((bw, C), lambda j: (j, 0))],
        out_specs=pl.BlockSpec((C, bw), lambda j: (0, j)),
        out_shape=jax.ShapeDtypeStruct((C, B), jnp.float32),
    )(rows)


def _sc_body(table, idxh, wgth, out_hbm,
             idxv, wgtv, r0, r1, o0, o1,
             wm00, wm10, wm01, wm11,
             sg0, sg1, so0, so1):
    wid = lax.axis_index("s") * NC + lax.axis_index("c")
    base = wid * PPW
    rbuf = (r0, r1)
    obuf = (o0, o1)
    sg = (sg0, sg1)
    so = (so0, so1)

    pltpu.sync_copy(idxh.at[pl.ds(base * 4, PPW * 4)], idxv)
    pltpu.sync_copy(wgth.at[pl.ds(base * 4, PPW * 4)], wgtv)

    def fire(b, g):
        off4 = pl.multiple_of(g * (4 * K), 4 * K)
        pltpu.async_copy(table.at[idxv.at[pl.ds(off4, 4 * K)]], rbuf[b], sg[b])

    def drain_gather(b):
        pltpu.make_async_copy(table.at[pl.ds(0, 4 * K)], rbuf[b], sg[b]).wait()

    def drain_out(b):
        pltpu.make_async_copy(out_hbm.at[pl.ds(0, K)], obuf[b], so[b]).wait()

    fire(0, 0)
    fire(1, 1)

    def loop2(g2, carry):
        for b in range(2):
            g = g2 * 2 + b
            off4 = pl.multiple_of(g * (4 * K), 4 * K)
            wv00 = wgtv[pl.ds(off4, 16)]
            wv10 = wgtv[pl.ds(off4 + 16, 16)]
            wv01 = wgtv[pl.ds(off4 + 32, 16)]
            wv11 = wgtv[pl.ds(off4 + 48, 16)]
            for l in range(16):
                wm00[l, :] = jnp.full((16,), wv00[l], jnp.float32)
                wm10[l, :] = jnp.full((16,), wv10[l], jnp.float32)
                wm01[l, :] = jnp.full((16,), wv01[l], jnp.float32)
                wm11[l, :] = jnp.full((16,), wv11[l], jnp.float32)
            drain_gather(b)
            with jax.named_scope("drain_out"):
                @pl.when(g2 > 0)
                def _():
                    drain_out(b)
            rb = rbuf[b]
            ob = obuf[b]

            def pix(k, carry2):
                w00b = wm00[k, :]
                w10b = wm10[k, :]
                w01b = wm01[k, :]
                w11b = wm11[k, :]
                for j in range(C // 16):
                    sl = pl.ds(16 * j, 16)
                    ob[k, sl] = (rb[k, sl] * w00b + rb[k + 16, sl] * w10b
                                 + rb[k + 32, sl] * w01b + rb[k + 48, sl] * w11b)
                return carry2

            lax.fori_loop(0, K, pix, 0)
            pltpu.async_copy(ob, out_hbm.at[pl.ds(base + g * K, K)], so[b])
            with jax.named_scope("refire"):
                @pl.when(g2 < NCH // 2 - 1)
                def _():
                    fire(b, g + 2)
        return carry

    lax.fori_loop(0, NCH // 2, loop2, 0)
    drain_out(0)
    drain_out(1)


@functools.lru_cache(maxsize=1)
def _sc_gather():
  return functools.partial(
    pl.kernel,
    out_type=jax.ShapeDtypeStruct((B, C), jnp.float32),
    mesh=plsc.VectorSubcoreMesh(core_axis_name="c", subcore_axis_name="s",
                                num_cores=NC, num_subcores=NS),
    scratch_types=[
        pltpu.VMEM((PPW * 4,), jnp.int32),
        pltpu.VMEM((PPW * 4,), jnp.float32),
        pltpu.VMEM((4 * K, C), jnp.float32),
        pltpu.VMEM((4 * K, C), jnp.float32),
        pltpu.VMEM((K, C), jnp.float32),
        pltpu.VMEM((K, C), jnp.float32),
        pltpu.VMEM((16, 16), jnp.float32),
        pltpu.VMEM((16, 16), jnp.float32),
        pltpu.VMEM((16, 16), jnp.float32),
        pltpu.VMEM((16, 16), jnp.float32),
        pltpu.SemaphoreType.DMA,
        pltpu.SemaphoreType.DMA,
        pltpu.SemaphoreType.DMA,
        pltpu.SemaphoreType.DMA,
    ],
  )(_sc_body)


def kernel(uv_map, neural_tex):
    u = uv_map[0, :, :, 0]
    v = uv_map[0, :, :, 1]
    i00, i10, i01, i11, w00, w10, w01, w11 = _prep(u, v)
    idx4 = jnp.stack([i00.reshape(B), i10.reshape(B),
                      i01.reshape(B), i11.reshape(B)])
    wgt4 = jnp.stack([w00.reshape(B), w10.reshape(B),
                      w01.reshape(B), w11.reshape(B)])
    idx_flat = idx4.reshape(4, B // K, K).transpose(1, 0, 2).reshape(-1)
    wgt_flat = wgt4.reshape(4, B // K, K).transpose(1, 0, 2).reshape(-1)
    table = _transpose_in(neural_tex.reshape(C, B))
    rows = _sc_gather()(table, idx_flat, wgt_flat)
    out = _transpose_out(rows)
    return out.reshape(1, C, H, W)


# speedup vs baseline: 2.7734x; 1.2948x over previous
import functools

import jax
import jax.numpy as jnp
from jax import lax
from jax.experimental import pallas as pl
from jax.experimental.pallas import tpu as pltpu
from jax.experimental.pallas import tpu_sc as plsc

H = W = 384
C = 384
B = H * W
NC, NS = 2, 16
NW = NC * NS
PPW = B // NW
K = 16
NCH = PPW // K


def _prep_body(u_ref, v_ref, i00, i10, i01, i11, w00, w10, w01, w11):
    u = u_ref[...]
    v = v_ref[...]
    gx = u * 2.0 - 1.0
    gy = -(v * 2.0 - 1.0)
    ix = ((gx + 1.0) * W - 1.0) * 0.5
    iy = ((gy + 1.0) * H - 1.0) * 0.5
    ix0 = jnp.floor(ix)
    iy0 = jnp.floor(iy)
    ix1 = ix0 + 1.0
    iy1 = iy0 + 1.0
    wx1 = ix - ix0
    wx0 = 1.0 - wx1
    wy1 = iy - iy0
    wy0 = 1.0 - wy1
    vx0 = (ix0 >= 0) & (ix0 <= W - 1)
    vx1 = (ix1 >= 0) & (ix1 <= W - 1)
    vy0 = (iy0 >= 0) & (iy0 <= H - 1)
    vy1 = (iy1 >= 0) & (iy1 <= H - 1)
    cx0 = jnp.clip(ix0, 0, W - 1).astype(jnp.int32)
    cx1 = jnp.clip(ix1, 0, W - 1).astype(jnp.int32)
    cy0 = jnp.clip(iy0, 0, H - 1).astype(jnp.int32)
    cy1 = jnp.clip(iy1, 0, H - 1).astype(jnp.int32)
    live = u != 0.0
    zero = jnp.zeros_like(u)
    i00[...] = cy0 * W + cx0
    i10[...] = cy0 * W + cx1
    i01[...] = cy1 * W + cx0
    i11[...] = cy1 * W + cx1
    w00[...] = jnp.where(vx0 & vy0 & live, wx0 * wy0, zero)
    w10[...] = jnp.where(vx1 & vy0 & live, wx1 * wy0, zero)
    w01[...] = jnp.where(vx0 & vy1 & live, wx0 * wy1, zero)
    w11[...] = jnp.where(vx1 & vy1 & live, wx1 * wy1, zero)


def _prep(u, v):
    shp = jax.ShapeDtypeStruct((H, W), jnp.int32)
    shpf = jax.ShapeDtypeStruct((H, W), jnp.float32)
    return pl.pallas_call(
        _prep_body,
        out_shape=(shp, shp, shp, shp, shpf, shpf, shpf, shpf),
    )(u, v)


def _tr_body(in_ref, out_ref):
    out_ref[...] = in_ref[...].T


def _transpose_in(tex2):
    bw = 2048
    return pl.pallas_call(
        _tr_body,
        grid=(B // bw,),
        in_specs=[pl.BlockSpec((C, bw), lambda j: (0, j))],
        out_specs=pl.BlockSpec((bw, C), lambda j: (j, 0)),
        out_shape=jax.ShapeDtypeStruct((B, C), jnp.float32),
    )(tex2)


def _transpose_out(rows):
    bw = 2048
    return pl.pallas_call(
        _tr_body,
        grid=(B // bw,),
        in_specs=[pl.BlockSpec((bw, C), lambda j: (j, 0))],
        out_specs=pl.BlockSpec((C, bw), lambda j: (0, j)),
        out_shape=jax.ShapeDtypeStruct((C, B), jnp.float32),
    )(rows)


def _sc_body(table, idxh, wgth, out_hbm,
             idxv, wgtv, r0, r1, o0, o1,
             wm00, wm10, wm01, wm11,
             sg0, sg1, so0, so1):
    wid = lax.axis_index("s") * NC + lax.axis_index("c")
    base = wid * PPW
    rbuf = (r0, r1)
    obuf = (o0, o1)
    sg = (sg0, sg1)
    so = (so0, so1)

    pltpu.sync_copy(idxh.at[pl.ds(base * 4, PPW * 4)], idxv)
    pltpu.sync_copy(wgth.at[pl.ds(base * 4, PPW * 4)], wgtv)

    def fire(b, g):
        off4 = pl.multiple_of(g * (4 * K), 4 * K)
        pltpu.async_copy(table.at[idxv.at[pl.ds(off4, 4 * K)]], rbuf[b], sg[b])

    def drain_gather(b):
        pltpu.make_async_copy(table.at[pl.ds(0, 4 * K)], rbuf[b], sg[b]).wait()

    def drain_out(b):
        pltpu.make_async_copy(out_hbm.at[pl.ds(0, K)], obuf[b], so[b]).wait()

    fire(0, 0)
    fire(1, 1)

    def loop2(g2, carry):
        for b in range(2):
            g = g2 * 2 + b
            off4 = pl.multiple_of(g * (4 * K), 4 * K)
            wv00 = wgtv[pl.ds(off4, 16)]
            wv10 = wgtv[pl.ds(off4 + 16, 16)]
            wv01 = wgtv[pl.ds(off4 + 32, 16)]
            wv11 = wgtv[pl.ds(off4 + 48, 16)]
            for l in range(16):
                wm00[l, :] = jnp.full((16,), wv00[l], jnp.float32)
                wm10[l, :] = jnp.full((16,), wv10[l], jnp.float32)
                wm01[l, :] = jnp.full((16,), wv01[l], jnp.float32)
                wm11[l, :] = jnp.full((16,), wv11[l], jnp.float32)
            drain_gather(b)
            with jax.named_scope("drain_out"):
                @pl.when(g2 > 0)
                def _():
                    drain_out(b)
            rb = rbuf[b]
            ob = obuf[b]

            def pix(k, carry2):
                w00b = wm00[k, :]
                w10b = wm10[k, :]
                w01b = wm01[k, :]
                w11b = wm11[k, :]
                for j in range(C // 16):
                    sl = pl.ds(16 * j, 16)
                    ob[k, sl] = (rb[k, sl] * w00b + rb[k + 16, sl] * w10b
                                 + rb[k + 32, sl] * w01b + rb[k + 48, sl] * w11b)
                return carry2

            lax.fori_loop(0, K, pix, 0)
            pltpu.async_copy(ob, out_hbm.at[pl.ds(base + g * K, K)], so[b])
            with jax.named_scope("refire"):
                @pl.when(g2 < NCH // 2 - 1)
                def _():
                    fire(b, g + 2)
        return carry

    lax.fori_loop(0, NCH // 2, loop2, 0)
    drain_out(0)
    drain_out(1)


@functools.lru_cache(maxsize=1)
def _sc_gather():
  return functools.partial(
    pl.kernel,
    out_type=jax.ShapeDtypeStruct((B, C), jnp.float32),
    mesh=plsc.VectorSubcoreMesh(core_axis_name="c", subcore_axis_name="s",
                                num_cores=NC, num_subcores=NS),
    scratch_types=[
        pltpu.VMEM((PPW * 4,), jnp.int32),
        pltpu.VMEM((PPW * 4,), jnp.float32),
        pltpu.VMEM((4 * K, C), jnp.float32),
        pltpu.VMEM((4 * K, C), jnp.float32),
        pltpu.VMEM((K, C), jnp.float32),
        pltpu.VMEM((K, C), jnp.float32),
        pltpu.VMEM((16, 16), jnp.float32),
        pltpu.VMEM((16, 16), jnp.float32),
        pltpu.VMEM((16, 16), jnp.float32),
        pltpu.VMEM((16, 16), jnp.float32),
        pltpu.SemaphoreType.DMA,
        pltpu.SemaphoreType.DMA,
        pltpu.SemaphoreType.DMA,
        pltpu.SemaphoreType.DMA,
    ],
  )(_sc_body)


def kernel(uv_map, neural_tex):
    u = uv_map[0, :, :, 0]
    v = uv_map[0, :, :, 1]
    i00, i10, i01, i11, w00, w10, w01, w11 = _prep(u, v)
    idx4 = jnp.stack([i00.reshape(B), i10.reshape(B),
                      i01.reshape(B), i11.reshape(B)])
    wgt4 = jnp.stack([w00.reshape(B), w10.reshape(B),
                      w01.reshape(B), w11.reshape(B)])
    idx_flat = idx4.reshape(4, B // K, K).transpose(1, 0, 2).reshape(-1)
    wgt_flat = wgt4.reshape(4, B // K, K).transpose(1, 0, 2).reshape(-1)
    table = jnp.transpose(neural_tex.reshape(C, B))
    rows = _sc_gather()(table, idx_flat, wgt_flat)
    out = jnp.transpose(rows)
    return out.reshape(1, C, H, W)
